# Initial kernel scaffold; baseline (speedup 1.0000x reference)
import jax
import jax.numpy as jnp
from jax.experimental import pallas as pl


def kernel(x, coords, W1, g1, b1, W2, g2, b2, W3, g3, b3, W4, g4, b4, dW1, dg1, db1, fW, cW1, cg1, cb1, cW2):
    raise NotImplementedError("write your pallas kernel here")


# speedup vs baseline: 8.5950x; 8.5950x over previous
import functools

import jax
import jax.numpy as jnp
from jax import lax
from jax.experimental import pallas as pl
from jax.experimental.pallas import tpu as pltpu
from jax.experimental.pallas import tpu_sc as plsc

B = 16
N = 1024
BN = B * N
KNN = 9
EPS = 1e-5
NW = 32
NPW = BN // NW


def _knn_body(x_ref, out_ref, d_ref):
    b = pl.program_id(0)
    xb = x_ref[0]
    gram = lax.dot_general(xb, xb, (((0,), (0,)), ((), ())),
                           preferred_element_type=jnp.float32)
    sq = jnp.sum(xb * xb, axis=0)
    row = lax.broadcasted_iota(jnp.int32, (N, N), 0)
    col = lax.broadcasted_iota(jnp.int32, (N, N), 1)
    d = sq[None, :] - 2.0 * gram
    d_ref[...] = jnp.where(row == col, jnp.float32(1e30), d)
    base = b * N
    for j in range(KNN):
        d = d_ref[...]
        rmin = jnp.min(d, axis=1, keepdims=True)
        am = jnp.min(jnp.where(d == rmin, col, N), axis=1)
        out_ref[0, j:j + 1, :] = (am + base)[None, :]
        d_ref[...] = jnp.where(col == am[:, None], jnp.float32(1e30), d)


def _knn(x):
    return pl.pallas_call(
        _knn_body,
        grid=(B,),
        in_specs=[pl.BlockSpec((1, 3, N), lambda b: (b, 0, 0))],
        out_specs=pl.BlockSpec((1, KNN, N), lambda b: (b, 0, 0)),
        out_shape=jax.ShapeDtypeStruct((B, KNN, N), jnp.int32),
        scratch_shapes=[pltpu.VMEM((N, N), jnp.float32)],
    )(x)


@functools.lru_cache(maxsize=None)
def _make_sc_dx(c, cn):
    nchunks = NPW // cn
    mesh = plsc.VectorSubcoreMesh(core_axis_name="c", subcore_axis_name="s",
                                  num_cores=2, num_subcores=16)

    @functools.partial(
        pl.kernel,
        out_type=jax.ShapeDtypeStruct((KNN, BN, c), jnp.float32),
        mesh=mesh,
        scratch_types=[
            pltpu.VMEM((KNN * NPW,), jnp.int32),
            pltpu.VMEM((KNN, cn, c), jnp.float32),
            pltpu.VMEM((cn, c), jnp.float32),
            pltpu.SemaphoreType.DMA,
            pltpu.SemaphoreType.DMA,
        ],
    )
    def sc_kernel(z_hbm, idx_hbm, dx_hbm, idx_v, rows_v, xi_v, gsem, xsem):
        cid = lax.axis_index("c")
        sid = lax.axis_index("s")
        wid = sid * 2 + cid
        b = wid // 2
        half = wid % 2
        gbase = wid * NPW
        for k in range(KNN):
            pltpu.sync_copy(
                idx_hbm.at[pl.ds((b * KNN + k) * N + half * NPW, NPW)],
                idx_v.at[pl.ds(k * NPW, NPW)])

        def chunk_body(ci, carry):
            base = gbase + ci * cn
            xcp = pltpu.async_copy(z_hbm.at[pl.ds(base, cn)], xi_v, xsem)
            cps = []
            for k in range(KNN):
                cps.append(pltpu.async_copy(
                    z_hbm.at[idx_v.at[pl.ds(k * NPW + ci * cn, cn)]],
                    rows_v.at[k], gsem))
            xcp.wait()
            for cp in cps:
                cp.wait()

            def node_body(i, carry2):
                for oc in range(c // 16):
                    sl = pl.ds(oc * 16, 16)
                    xv = xi_v[i, sl]
                    for k in range(KNN):
                        rows_v[k, i, sl] = rows_v[k, i, sl] - xv
                return carry2

            lax.fori_loop(0, cn, node_body, 0)
            for k in range(KNN):
                pltpu.sync_copy(rows_v.at[k], dx_hbm.at[k, pl.ds(base, cn)])
            return carry

        lax.fori_loop(0, nchunks, chunk_body, 0)

    return sc_kernel


_SC_CHUNK = {128: 64, 256: 32}


_RT = 512
_NT = BN // _RT


def _edge_body(z_ref, dx_ref, wa_ref, wb_ref, g_ref, bb_ref,
               pm_ref, scale_ref, shift_ref, p_sc, m_sc, s_sc, acc_ref):
    i = pl.program_id(0)
    k = pl.program_id(1)
    dn = (((1,), (1,)), ((), ()))
    y = lax.dot_general(dx_ref[0], wb_ref[...], dn,
                        preferred_element_type=jnp.float32)

    @pl.when(jnp.logical_and(i == 0, k == 0))
    def _():
        acc_ref[...] = jnp.zeros_like(acc_ref)

    @pl.when(k == 0)
    def _():
        p_sc[...] = lax.dot_general(z_ref[...], wa_ref[...], dn,
                                    preferred_element_type=jnp.float32)
        m_sc[...] = y
        s_sc[...] = y

    @pl.when(k > 0)
    def _():
        m_sc[...] = jnp.maximum(m_sc[...], y)
        s_sc[...] = s_sc[...] + y

    acc_ref[4:5] += jnp.sum(y * y, axis=0, keepdims=True)

    @pl.when(k == KNN - 1)
    def _():
        p = p_sc[...]
        s = s_sc[...]
        pm_ref[...] = p + m_sc[...]
        acc_ref[0:1] += jnp.sum(p, axis=0, keepdims=True)
        acc_ref[1:2] += jnp.sum(p * p, axis=0, keepdims=True)
        acc_ref[2:3] += jnp.sum(s, axis=0, keepdims=True)
        acc_ref[3:4] += jnp.sum(p * s, axis=0, keepdims=True)

    @pl.when(jnp.logical_and(i == _NT - 1, k == KNN - 1))
    def _():
        e = float(BN * KNN)
        mean = (KNN * acc_ref[0:1] + acc_ref[2:3]) / e
        ey2 = (KNN * acc_ref[1:2] + 2.0 * acc_ref[3:4] + acc_ref[4:5]) / e
        var = ey2 - mean * mean
        inv = lax.rsqrt(var + EPS)
        scale = g_ref[...] * inv
        scale_ref[...] = scale
        shift_ref[...] = bb_ref[...] - mean * scale


def _edge_mm(z, dx, wa, wb, g, bb):
    o, c = wa.shape
    return pl.pallas_call(
        _edge_body,
        grid=(_NT, KNN),
        in_specs=[pl.BlockSpec((_RT, c), lambda i, k: (i, 0)),
                  pl.BlockSpec((1, _RT, c), lambda i, k: (k, i, 0)),
                  pl.BlockSpec((o, c), lambda i, k: (0, 0)),
                  pl.BlockSpec((o, c), lambda i, k: (0, 0)),
                  pl.BlockSpec((1, o), lambda i, k: (0, 0)),
                  pl.BlockSpec((1, o), lambda i, k: (0, 0))],
        out_specs=[pl.BlockSpec((_RT, o), lambda i, k: (i, 0)),
                   pl.BlockSpec((1, o), lambda i, k: (0, 0)),
                   pl.BlockSpec((1, o), lambda i, k: (0, 0))],
        out_shape=[jax.ShapeDtypeStruct((BN, o), jnp.float32),
                   jax.ShapeDtypeStruct((1, o), jnp.float32),
                   jax.ShapeDtypeStruct((1, o), jnp.float32)],
        scratch_shapes=[pltpu.VMEM((_RT, o), jnp.float32),
                        pltpu.VMEM((_RT, o), jnp.float32),
                        pltpu.VMEM((_RT, o), jnp.float32),
                        pltpu.VMEM((8, o), jnp.float32)],
    )(z, dx, wa, wb, g.reshape(1, o), bb.reshape(1, o))


def _norm_body(pm_ref, scale_ref, shift_ref, z_ref):
    z_ref[...] = jnp.maximum(
        pm_ref[...] * scale_ref[...] + shift_ref[...], 0.0)


def _norm(pm, scale, shift, tile=2048):
    o = pm.shape[1]
    return pl.pallas_call(
        _norm_body,
        grid=(BN // tile,),
        in_specs=[pl.BlockSpec((tile, o), lambda i: (i, 0)),
                  pl.BlockSpec((1, o), lambda i: (0, 0)),
                  pl.BlockSpec((1, o), lambda i: (0, 0))],
        out_specs=pl.BlockSpec((tile, o), lambda i: (i, 0)),
        out_shape=jax.ShapeDtypeStruct((BN, o), jnp.float32),
    )(pm, scale, shift)


_NT_STATS = 16


def _d1_body(z1_ref, w_ref, g_ref, bb_ref, u_ref, us_ref, ush_ref, acc_ref):
    i = pl.program_id(0)

    @pl.when(i == 0)
    def _():
        acc_ref[...] = jnp.zeros_like(acc_ref)

    u = lax.dot_general(z1_ref[...], w_ref[...], (((1,), (1,)), ((), ())),
                        preferred_element_type=jnp.float32)
    u_ref[...] = u
    acc_ref[0:1] += jnp.sum(u, axis=0, keepdims=True)
    acc_ref[1:2] += jnp.sum(u * u, axis=0, keepdims=True)

    @pl.when(i == _NT_STATS - 1)
    def _():
        mu = acc_ref[0:1] / float(BN)
        var = acc_ref[1:2] / float(BN) - mu * mu
        inv = lax.rsqrt(var + EPS)
        sc = g_ref[...] * inv
        us_ref[...] = sc
        ush_ref[...] = bb_ref[...] - mu * sc


def _d1(z1, dw, g, bb):
    o = dw.shape[0]
    tile = BN // _NT_STATS
    return pl.pallas_call(
        _d1_body,
        grid=(_NT_STATS,),
        in_specs=[pl.BlockSpec((tile, 128), lambda i: (i, 0)),
                  pl.BlockSpec((o, 128), lambda i: (0, 0)),
                  pl.BlockSpec((1, o), lambda i: (0, 0)),
                  pl.BlockSpec((1, o), lambda i: (0, 0))],
        out_specs=[pl.BlockSpec((tile, o), lambda i: (i, 0)),
                   pl.BlockSpec((1, o), lambda i: (0, 0)),
                   pl.BlockSpec((1, o), lambda i: (0, 0))],
        out_shape=[jax.ShapeDtypeStruct((BN, o), jnp.float32),
                   jax.ShapeDtypeStruct((1, o), jnp.float32),
                   jax.ShapeDtypeStruct((1, o), jnp.float32)],
        scratch_shapes=[pltpu.VMEM((2, o), jnp.float32)],
    )(z1, dw, g.reshape(1, o), bb.reshape(1, o))


def _d2_body(u_ref, us_ref, ush_ref, z4_ref, fwa_ref, fwb_ref, x1_ref, x2_ref):
    llf = jnp.maximum(u_ref[...] * us_ref[...] + ush_ref[...], 0.0)
    dn = (((1,), (1,)), ((), ()))
    fusion = (lax.dot_general(z4_ref[...], fwa_ref[...], dn,
                              preferred_element_type=jnp.float32) +
              lax.dot_general(llf, fwb_ref[...], dn,
                              preferred_element_type=jnp.float32))
    x1_ref[...] = jnp.max(fusion, axis=0, keepdims=True)[None]
    x2_ref[...] = (jnp.sum(fusion, axis=0, keepdims=True) * (1.0 / N))[None]


def _d2(u, us, ush, z4, fwa, fwb):
    return pl.pallas_call(
        _d2_body,
        grid=(B,),
        in_specs=[pl.BlockSpec((N, 48), lambda b: (b, 0)),
                  pl.BlockSpec((1, 48), lambda b: (0, 0)),
                  pl.BlockSpec((1, 48), lambda b: (0, 0)),
                  pl.BlockSpec((N, 256), lambda b: (b, 0)),
                  pl.BlockSpec((256, 256), lambda b: (0, 0)),
                  pl.BlockSpec((256, 48), lambda b: (0, 0))],
        out_specs=[pl.BlockSpec((1, 1, 256), lambda b: (b, 0, 0)),
                   pl.BlockSpec((1, 1, 256), lambda b: (b, 0, 0))],
        out_shape=[jax.ShapeDtypeStruct((B, 1, 256), jnp.float32)] * 2,
    )(u, us, ush, z4, fwa, fwb)


def _d3_body(x1_ref, x2_ref, c1a_ref, c1b_ref, cg_ref, cb_ref, c2_ref,
             out_ref):
    dn = (((1,), (1,)), ((), ()))
    v = (lax.dot_general(x1_ref[...], c1a_ref[...], dn,
                         preferred_element_type=jnp.float32) +
         lax.dot_general(x2_ref[...], c1b_ref[...], dn,
                         preferred_element_type=jnp.float32))
    mv = jnp.mean(v, axis=0, keepdims=True)
    vv = jnp.mean(v * v, axis=0, keepdims=True) - mv * mv
    h = jnp.maximum(
        (v - mv) * lax.rsqrt(vv + EPS) * cg_ref[...] + cb_ref[...], 0.0)
    out_ref[...] = lax.dot_general(h, c2_ref[...], dn,
                                   preferred_element_type=jnp.float32)


def _d3(x1, x2, c1a, c1b, cg, cb, c2):
    return pl.pallas_call(
        _d3_body,
        in_specs=[pl.BlockSpec(x1.shape, lambda: (0, 0)),
                  pl.BlockSpec(x2.shape, lambda: (0, 0)),
                  pl.BlockSpec(c1a.shape, lambda: (0, 0)),
                  pl.BlockSpec(c1b.shape, lambda: (0, 0)),
                  pl.BlockSpec((1, 256), lambda: (0, 0)),
                  pl.BlockSpec((1, 256), lambda: (0, 0)),
                  pl.BlockSpec(c2.shape, lambda: (0, 0))],
        out_specs=pl.BlockSpec((B, 40), lambda: (0, 0)),
        out_shape=jax.ShapeDtypeStruct((B, 40), jnp.float32),
    )(x1, x2, c1a, c1b, cg.reshape(1, 256), cb.reshape(1, 256), c2)


def _edge_layer(z, idx_flat, wa, wb, g, bb):
    c = z.shape[1]
    dx = _make_sc_dx(c, _SC_CHUNK[c])(z, idx_flat)
    pm, scale, shift = _edge_mm(z, dx, wa, wb, g, bb)
    return _norm(pm, scale, shift)


def _pad_w(w, o_pad, c_pad):
    o, c = w.shape
    return jnp.pad(w, ((0, o_pad - o), (0, c_pad - c)))


def kernel(x, coords, W1, g1, b1, W2, g2, b2, W3, g3, b3, W4, g4, b4,
           dW1, dg1, db1, fW, cW1, cg1, cb1, cW2):
    idx = _knn(x)
    idx_flat = idx.reshape(-1)
    z0 = jnp.pad(jnp.transpose(x, (0, 2, 1)).reshape(BN, 3),
                 ((0, 0), (0, 125)))
    zf = jnp.zeros((64,), jnp.float32)
    z1 = _edge_layer(z0, idx_flat,
                     _pad_w(W1[:, :3], 128, 128), _pad_w(W1[:, 3:], 128, 128),
                     jnp.concatenate([g1, zf]), jnp.concatenate([b1, zf]))
    z2 = _edge_layer(z1, idx_flat,
                     _pad_w(W2[:, :64], 128, 128), _pad_w(W2[:, 64:], 128, 128),
                     g2, b2)
    z3 = _edge_layer(z2, idx_flat, W3[:, :128], W3[:, 128:], g3, b3)
    z4 = _edge_layer(z3, idx_flat, W4[:, :256], W4[:, 256:], g4, b4)
    dw1p = jnp.concatenate([dW1, jnp.zeros((48, 64), jnp.float32)], axis=1)
    u, us, ush = _d1(z1, dw1p, dg1, db1)
    x1, x2 = _d2(u, us, ush, z4, fW[:, :256], fW[:, 256:])
    x1 = x1.reshape(B, 256)
    x2 = x2.reshape(B, 256)
    logits = _d3(x1, x2, cW1[:, :256], cW1[:, 256:], cg1, cb1, cW2)
    return logits
